# speedup vs baseline: 1.0055x; 1.0055x over previous
import jax
import jax.numpy as jnp
from jax.experimental import pallas as pl

N = 50000
H = 64
OUT = 2


def _head_body(h_ref, wh_ref, bh_ref, o_ref):
    o_ref[...] = h_ref[...] @ wh_ref[...] + bh_ref[...]


def _head(h, Wh, bh):
    BR = 2000
    return pl.pallas_call(
        _head_body,
        grid=(N // BR,),
        in_specs=[
            pl.BlockSpec((BR, H), lambda i: (i, 0)),
            pl.BlockSpec((H, OUT), lambda i: (0, 0)),
            pl.BlockSpec((OUT,), lambda i: (0,)),
        ],
        out_specs=pl.BlockSpec((BR, OUT), lambda i: (i, 0)),
        out_shape=jax.ShapeDtypeStruct((N, OUT), jnp.float32),
    )(h, Wh, bh)


def _layer(h, pos, src, dst, Wa, ba, Wb, bb):
    edge_feat = jnp.concatenate([h[src], pos[src] - pos[dst]], axis=-1)
    m = jnp.maximum(edge_feat @ Wa + ba, 0.0) @ Wb + bb
    out = jax.ops.segment_max(m, dst, num_segments=N)
    return jnp.where(jnp.isneginf(out), 0.0, out)


def kernel(x, pos, edge_index, W0a, b0a, W0b, b0b, W1a, b1a, W1b, b1b,
           W2a, b2a, W2b, b2b, Wh, bh):
    src = edge_index[0]
    dst = edge_index[1]
    h = _layer(x, pos, src, dst, W0a, b0a, W0b, b0b)
    h = jnp.maximum(h, 0.0)
    h = _layer(h, pos, src, dst, W1a, b1a, W1b, b1b)
    h = jnp.maximum(h, 0.0)
    h = _layer(h, pos, src, dst, W2a, b2a, W2b, b2b)
    h = jnp.maximum(h, 0.0)
    return _head(h, Wh, bh)


# speedup vs baseline: 1.7753x; 1.7656x over previous
import functools

import jax
import jax.numpy as jnp
from jax import lax
from jax.experimental import pallas as pl
from jax.experimental.pallas import tpu as pltpu
from jax.experimental.pallas import tpu_sc as plsc

N = 50000
E = 800000
H = 64
OUT = 2

NC = 2
NS = 16
NW = NC * NS

CH = 512
EP = 802816
EC = EP // NW
NCHUNK = EC // CH
IB = CH // 128

_mesh = plsc.VectorSubcoreMesh(
    core_axis_name="c", subcore_axis_name="s", num_cores=NC, num_subcores=NS)


def _s1_body(g_hbm, np_hbm, src_hbm, dst_hbm, out_hbm,
             src_v, dst_v, g_v, p_v, sem):
    wid = lax.axis_index("s") * NC + lax.axis_index("c")
    rbase = wid * (EC // 128)

    def chunk(i, _):
        roff = rbase + i * IB
        pltpu.sync_copy(src_hbm.at[pl.ds(roff, IB)], src_v)
        pltpu.sync_copy(dst_hbm.at[pl.ds(roff, IB)], dst_v)
        descs = []
        for q in range(IB):
            descs.append(pltpu.async_copy(
                g_hbm.at[src_v.at[q]], g_v.at[pl.ds(q * 128, 128)], sem))
            descs.append(pltpu.async_copy(
                np_hbm.at[dst_v.at[q]], p_v.at[pl.ds(q * 128, 128)], sem))
        for d in descs:
            d.wait()

        def sub_row(j, _):
            for c in range(H // 16):
                sl = pl.ds(c * 16, 16)
                g_v[j, sl] = g_v[j, sl] + p_v[j, sl]
            return _

        lax.fori_loop(0, CH, sub_row, None)
        pltpu.sync_copy(g_v, out_hbm.at[pl.ds(roff * 128, CH)])
        return _

    lax.fori_loop(0, NCHUNK, chunk, None)


@functools.partial(jax.jit, static_argnames=())
def _s1(G, negP, src2d, dst2d):
    return pl.kernel(
        _s1_body,
        out_type=jax.ShapeDtypeStruct((EP, H), jnp.float32),
        mesh=_mesh,
        scratch_types=[
            pltpu.VMEM((IB, 128), jnp.int32),
            pltpu.VMEM((IB, 128), jnp.int32),
            pltpu.VMEM((CH, H), jnp.float32),
            pltpu.VMEM((CH, H), jnp.float32),
            pltpu.SemaphoreType.DMA,
        ],
        compiler_params=pltpu.CompilerParams(use_tc_tiling_on_sc=False),
    )(G, negP, src2d, dst2d)


def _head_body(h_ref, wh_ref, bh_ref, o_ref):
    o_ref[...] = h_ref[...] @ wh_ref[...] + bh_ref[...]


def _head(h, Wh, bh):
    BR = 2000
    return pl.pallas_call(
        _head_body,
        grid=(N // BR,),
        in_specs=[
            pl.BlockSpec((BR, H), lambda i: (i, 0)),
            pl.BlockSpec((H, OUT), lambda i: (0, 0)),
            pl.BlockSpec((OUT,), lambda i: (0,)),
        ],
        out_specs=pl.BlockSpec((BR, OUT), lambda i: (i, 0)),
        out_shape=jax.ShapeDtypeStruct((N, OUT), jnp.float32),
    )(h, Wh, bh)


def _layer(h, pos, src2d, dst2d, dst, Wa, ba, Wb, bb):
    hin = h.shape[1]
    G = h @ Wa[:hin] + pos @ Wa[hin:] + ba
    negP = -(pos @ Wa[hin:])
    pre = _s1(G, negP, src2d, dst2d)[:E]
    m = jnp.maximum(pre, 0.0) @ Wb + bb
    agg = jax.ops.segment_max(m, dst, num_segments=N)
    return jnp.maximum(agg, 0.0)


def kernel(x, pos, edge_index, W0a, b0a, W0b, b0b, W1a, b1a, W1b, b1b,
           W2a, b2a, W2b, b2b, Wh, bh):
    src = edge_index[0]
    dst = edge_index[1]
    src2d = jnp.pad(src, (0, EP - E)).reshape(EP // 128, 128)
    dst2d = jnp.pad(dst, (0, EP - E)).reshape(EP // 128, 128)
    h = _layer(x, pos, src2d, dst2d, dst, W0a, b0a, W0b, b0b)
    h = _layer(h, pos, src2d, dst2d, dst, W1a, b1a, W1b, b1b)
    h = _layer(h, pos, src2d, dst2d, dst, W2a, b2a, W2b, b2b)
    return _head(h, Wh, bh)
